# Initial kernel scaffold; baseline (speedup 1.0000x reference)
import jax
import jax.numpy as jnp
from jax.experimental import pallas as pl


def kernel(node_embed_weight, edge_index, W1, b1, W2, b2, Wc, bc):
    raise NotImplementedError("write your pallas kernel here")


# speedup vs baseline: 8.4585x; 8.4585x over previous
import functools

import jax
import jax.numpy as jnp
from jax import lax
from jax.experimental import pallas as pl
from jax.experimental.pallas import tpu as pltpu
from jax.experimental.pallas import tpu_sc as plsc

N = 10000
E = 160000
IN_DIM = 60
DP = 64
HID = 512
OUT2 = 32
NCLS = 16

NC, NS = 2, 16
NW = NC * NS
NPAD = 10240
NPT = NPAD // NS
CH = 128
EW = 5120
EPAD = NW * EW
NCH = EW // CH
RB = 256
NBLK = NPAD // RB
EPS = 1e-12

@functools.cache
def _mesh():
    return plsc.VectorSubcoreMesh(
        core_axis_name="c", subcore_axis_name="s",
        num_cores=NC, num_subcores=NS)


def _deg_body(srcp, dstp2, out, acc, idxb, ones, zbuf):
    c = lax.axis_index("c")
    s = lax.axis_index("s")
    gid = c * NS + s
    z16 = jnp.zeros((16,), jnp.float32)
    o16 = jnp.ones((16,), jnp.float32)
    zn = (2 * NPAD) // NS

    def zb(i, _):
        zbuf[pl.ds(i * 16, 16)] = z16
        return 0
    lax.fori_loop(0, zn // 16, zb, 0)

    def ob(i, _):
        ones[pl.ds(i * 16, 16)] = o16
        return 0
    lax.fori_loop(0, CH // 16, ob, 0)

    pltpu.sync_copy(zbuf, acc.at[pl.ds(s * zn, zn)])
    plsc.subcore_barrier()

    base = gid * EW

    def step(k, _):
        off = base + k * CH
        pltpu.sync_copy(srcp.at[pl.ds(off, CH)], idxb)
        pltpu.sync_copy(ones, acc.at[idxb], add=True)
        pltpu.sync_copy(dstp2.at[pl.ds(off, CH)], idxb)
        pltpu.sync_copy(ones, acc.at[idxb], add=True)
        return 0
    lax.fori_loop(0, NCH, step, 0)

    plsc.subcore_barrier()
    pltpu.sync_copy(acc.at[pl.ds(s * zn, zn)], zbuf)
    pltpu.sync_copy(zbuf, out.at[pl.ds(c * 2 * NPAD + s * zn, zn)])


_SC_PARAMS = pltpu.CompilerParams(use_tc_tiling_on_sc=False)


@functools.cache
def _deg_kernel():
    return pl.kernel(
        _deg_body,
        out_type=jax.ShapeDtypeStruct((2 * 2 * NPAD,), jnp.float32),
        mesh=_mesh(),
        compiler_params=_SC_PARAMS,
        scratch_types=[
            pltpu.VMEM_SHARED((2 * NPAD,), jnp.float32),
            pltpu.VMEM((CH,), jnp.int32),
            pltpu.VMEM((CH,), jnp.float32),
            pltpu.VMEM(((2 * NPAD) // NS,), jnp.float32),
        ],
    )


@functools.cache
def _make_agg(D):
    def body(table, srcp, dstp, out, acc, sidx, didx, rows):
        c = lax.axis_index("c")
        s = lax.axis_index("s")
        gid = c * NS + s
        z16 = jnp.zeros((16,), jnp.float32)

        def zr(i, _):
            def zc(j, _):
                rows[i, pl.ds(j * 16, 16)] = z16
                return 0
            return lax.fori_loop(0, D // 16, zc, 0)
        lax.fori_loop(0, CH, zr, 0)

        for kk in range(NPT // CH):
            pltpu.sync_copy(rows, acc.at[pl.ds(s * NPT + kk * CH, CH)])
        plsc.subcore_barrier()

        base = gid * EW

        def step(k, _):
            off = base + k * CH
            pltpu.sync_copy(srcp.at[pl.ds(off, CH)], sidx)
            pltpu.sync_copy(dstp.at[pl.ds(off, CH)], didx)
            pltpu.sync_copy(table.at[sidx], rows)
            pltpu.sync_copy(rows, acc.at[didx], add=True)
            return 0
        lax.fori_loop(0, NCH, step, 0)

        plsc.subcore_barrier()
        for kk in range(NPT // CH):
            r0 = s * NPT + kk * CH
            pltpu.sync_copy(acc.at[pl.ds(r0, CH)], rows)
            pltpu.sync_copy(rows, out.at[pl.ds(c * NPAD + r0, CH)])

    return pl.kernel(
        body,
        out_type=jax.ShapeDtypeStruct((2 * NPAD, D), jnp.float32),
        mesh=_mesh(),
        compiler_params=_SC_PARAMS,
        scratch_types=[
            pltpu.VMEM_SHARED((NPAD, D), jnp.float32),
            pltpu.VMEM((CH,), jnp.int32),
            pltpu.VMEM((CH,), jnp.int32),
            pltpu.VMEM((CH, D), jnp.float32),
        ],
    )


def _h0_body(x_ref, dgo_ref, o_ref):
    x = x_ref[...]
    dego = jnp.sum(dgo_ref[...], axis=0)
    norm_out = jnp.where(dego > 0, lax.rsqrt(dego), 0.0)
    xn = jnp.sqrt(jnp.sum(x * x, axis=-1))
    scale = norm_out / jnp.maximum(xn, EPS)
    o_ref[...] = x * scale[:, None]


def _h0_call(xp, degp):
    return pl.pallas_call(
        _h0_body,
        grid=(NBLK,),
        in_specs=[
            pl.BlockSpec((RB, DP), lambda i: (i, 0)),
            pl.BlockSpec((2, RB), lambda i: (0, i)),
        ],
        out_specs=pl.BlockSpec((RB, DP), lambda i: (i, 0)),
        out_shape=jax.ShapeDtypeStruct((NPAD, DP), jnp.float32),
    )(xp, degp)


def _l1_body(a0_ref, a1_ref, dgo_ref, dgi_ref, w1_ref, b1_ref, w2_ref, o_ref):
    dego = jnp.sum(dgo_ref[...], axis=0)
    degi = jnp.sum(dgi_ref[...], axis=0)
    no = jnp.where(dego > 0, lax.rsqrt(dego), 0.0)
    ni = jnp.where(degi > 0, lax.rsqrt(degi), 0.0)
    agg = (a0_ref[...] + a1_ref[...]) * ni[:, None]
    h = jnp.dot(agg, w1_ref[...], preferred_element_type=jnp.float32)
    h = h + b1_ref[...]
    hn = jnp.sqrt(jnp.sum(h * h, axis=-1, keepdims=True))
    h = h / jnp.maximum(hn, EPS)
    h = jnp.maximum(h, 0.0)
    hn = jnp.sqrt(jnp.sum(h * h, axis=-1, keepdims=True))
    h = h / jnp.maximum(hn, EPS)
    h = h * no[:, None]
    o_ref[...] = jnp.dot(h, w2_ref[...], preferred_element_type=jnp.float32)


def _l1_call(agg1, degp, w1p, b1r, w2):
    return pl.pallas_call(
        _l1_body,
        grid=(NBLK,),
        in_specs=[
            pl.BlockSpec((RB, DP), lambda i: (i, 0)),
            pl.BlockSpec((RB, DP), lambda i: (i + NBLK, 0)),
            pl.BlockSpec((2, RB), lambda i: (0, i)),
            pl.BlockSpec((2, RB), lambda i: (0, i + NBLK)),
            pl.BlockSpec((DP, HID), lambda i: (0, 0)),
            pl.BlockSpec((1, HID), lambda i: (0, 0)),
            pl.BlockSpec((HID, OUT2), lambda i: (0, 0)),
        ],
        out_specs=pl.BlockSpec((RB, OUT2), lambda i: (i, 0)),
        out_shape=jax.ShapeDtypeStruct((NPAD, OUT2), jnp.float32),
    )(agg1, agg1, degp, degp, w1p, b1r, w2)


def _l2_body(q0_ref, q1_ref, dgi_ref, b2_ref, wc_ref, bc_ref, o_ref, acc_ref):
    i = pl.program_id(0)
    degi = jnp.sum(dgi_ref[...], axis=0)
    ni = jnp.where(degi > 0, lax.rsqrt(degi), 0.0)
    h = (q0_ref[...] + q1_ref[...]) * ni[:, None] + b2_ref[...]
    hn = jnp.sqrt(jnp.sum(h * h, axis=-1, keepdims=True))
    h = h / jnp.maximum(hn, EPS)
    h = jnp.maximum(h, 0.0)
    hn = jnp.sqrt(jnp.sum(h * h, axis=-1, keepdims=True))
    h = h / jnp.maximum(hn, EPS)
    rid = lax.broadcasted_iota(jnp.int32, (RB, 1), 0) + i * RB
    h = jnp.where(rid < N, h, 0.0)
    part = jnp.sum(h, axis=0, keepdims=True)

    @pl.when(i == 0)
    def _():
        acc_ref[...] = part

    @pl.when(i > 0)
    def _():
        acc_ref[...] = acc_ref[...] + part

    @pl.when(i == NBLK - 1)
    def _():
        hg = acc_ref[...] / N
        hgn = jnp.sqrt(jnp.sum(hg * hg))
        hg = hg / jnp.maximum(hgn, EPS)
        o_ref[...] = (
            jnp.dot(hg, wc_ref[...], preferred_element_type=jnp.float32)
            + bc_ref[...])


def _l2_call(agg2, degp, b2r, wc, bcr):
    return pl.pallas_call(
        _l2_body,
        grid=(NBLK,),
        in_specs=[
            pl.BlockSpec((RB, OUT2), lambda i: (i, 0)),
            pl.BlockSpec((RB, OUT2), lambda i: (i + NBLK, 0)),
            pl.BlockSpec((2, RB), lambda i: (0, i + NBLK)),
            pl.BlockSpec((1, OUT2), lambda i: (0, 0)),
            pl.BlockSpec((OUT2, NCLS), lambda i: (0, 0)),
            pl.BlockSpec((1, NCLS), lambda i: (0, 0)),
        ],
        out_specs=pl.BlockSpec((1, NCLS), lambda i: (0, 0)),
        out_shape=jax.ShapeDtypeStruct((1, NCLS), jnp.float32),
        scratch_shapes=[pltpu.VMEM((1, OUT2), jnp.float32)],
        compiler_params=pltpu.CompilerParams(
            dimension_semantics=("arbitrary",)),
    )(agg2, agg2, degp, b2r, wc, bcr)


@jax.jit
def kernel(node_embed_weight, edge_index, W1, b1, W2, b2, Wc, bc):
    f32 = jnp.float32
    src = edge_index[0].astype(jnp.int32)
    dst = edge_index[1].astype(jnp.int32)
    npe = EPAD - E
    pad_idx = N + (jnp.arange(npe, dtype=jnp.int32) % (NPAD - N))
    srcp = jnp.concatenate([src, pad_idx])
    dstp = jnp.concatenate([dst, pad_idx])
    dstp2 = dstp + NPAD

    xp = jnp.zeros((NPAD, DP), f32).at[:N, :IN_DIM].set(
        node_embed_weight.astype(f32))
    w1p = jnp.zeros((DP, HID), f32).at[:IN_DIM].set(W1.astype(f32))

    degp = _deg_kernel()(srcp, dstp2).reshape(2, 2 * NPAD)
    h0s = _h0_call(xp, degp)
    agg1 = _make_agg(DP)(h0s, srcp, dstp)
    p = _l1_call(agg1, degp, w1p, b1.reshape(1, HID), W2)
    agg2 = _make_agg(OUT2)(p, srcp, dstp)
    return _l2_call(agg2, degp, b2.reshape(1, OUT2), Wc, bc.reshape(1, NCLS))


# speedup vs baseline: 14.9581x; 1.7684x over previous
import functools

import jax
import jax.numpy as jnp
from jax import lax
from jax.experimental import pallas as pl
from jax.experimental.pallas import tpu as pltpu
from jax.experimental.pallas import tpu_sc as plsc

N = 10000
E = 160000
IN_DIM = 60
DP = 64
HID = 512
OUT2 = 32
NCLS = 16

NC, NS = 2, 16
NW = NC * NS
NPAD = 10240
NPT = NPAD // NS
CH = 128
EW = 5120
EPAD = NW * EW
NCH = EW // CH
RB = 256
NBLK = NPAD // RB
EPS = 1e-12

@functools.cache
def _mesh():
    return plsc.VectorSubcoreMesh(
        core_axis_name="c", subcore_axis_name="s",
        num_cores=NC, num_subcores=NS)


def _deg_body(srcp2, dstp2, out, acc, sidx, didx, ones, zbuf, dsem):
    c = lax.axis_index("c")
    s = lax.axis_index("s")
    gid = c * NS + s
    z16 = jnp.zeros((16,), jnp.float32)
    o16 = jnp.ones((16,), jnp.float32)
    zn = (2 * NPAD) // NS

    def zb(i, _):
        zbuf[pl.ds(i * 16, 16)] = z16
        return 0
    lax.fori_loop(0, zn // 16, zb, 0)

    def ob(i, _):
        ones[pl.ds(i * 16, 16)] = o16
        return 0
    lax.fori_loop(0, CH // 16, ob, 0)

    pltpu.sync_copy(zbuf, acc.at[pl.ds(s * zn, zn)])
    plsc.subcore_barrier()

    pltpu.sync_copy(srcp2.at[pl.ds(gid * NCH, NCH)], sidx)
    pltpu.sync_copy(dstp2.at[pl.ds(gid * NCH, NCH)], didx)

    FD = 4

    def step(t, _):
        for j in range(FD):
            k = t * FD + j
            pltpu.async_copy(ones, acc.at[sidx.at[k]], dsem, add=True)
            pltpu.async_copy(ones, acc.at[didx.at[k]], dsem, add=True)
        for j in range(FD):
            k = t * FD + j
            pltpu.make_async_copy(ones, acc.at[sidx.at[k]], dsem).wait()
            pltpu.make_async_copy(ones, acc.at[didx.at[k]], dsem).wait()
        return 0
    lax.fori_loop(0, NCH // FD, step, 0)

    plsc.subcore_barrier()
    pltpu.sync_copy(acc.at[pl.ds(s * zn, zn)], zbuf)
    pltpu.sync_copy(zbuf, out.at[pl.ds(c * 2 * NPAD + s * zn, zn)])


_SC_PARAMS = pltpu.CompilerParams(use_tc_tiling_on_sc=False)


@functools.cache
def _deg_kernel():
    return pl.kernel(
        _deg_body,
        out_type=jax.ShapeDtypeStruct((2 * 2 * NPAD,), jnp.float32),
        mesh=_mesh(),
        compiler_params=_SC_PARAMS,
        scratch_types=[
            pltpu.VMEM_SHARED((2 * NPAD,), jnp.float32),
            pltpu.VMEM((NCH, CH), jnp.int32),
            pltpu.VMEM((NCH, CH), jnp.int32),
            pltpu.VMEM((CH,), jnp.float32),
            pltpu.VMEM(((2 * NPAD) // NS,), jnp.float32),
            pltpu.SemaphoreType.DMA,
        ],
    )


NBUF = 4


@functools.cache
def _make_agg(D):
    def body(table, srcp2, dstp2, out, acc, sidx, didx, rows, gsem, ssem):
        c = lax.axis_index("c")
        s = lax.axis_index("s")
        gid = c * NS + s
        z16 = jnp.zeros((16,), jnp.float32)

        def zr(i, _):
            def zc(j, _):
                rows[0, i, pl.ds(j * 16, 16)] = z16
                return 0
            return lax.fori_loop(0, D // 16, zc, 0)
        lax.fori_loop(0, CH, zr, 0)

        for kk in range(NPT // CH):
            pltpu.sync_copy(rows.at[0], acc.at[pl.ds(s * NPT + kk * CH, CH)])
        plsc.subcore_barrier()

        pltpu.sync_copy(srcp2.at[pl.ds(gid * NCH, NCH)], sidx)
        pltpu.sync_copy(dstp2.at[pl.ds(gid * NCH, NCH)], didx)

        def g_start(k, b):
            pltpu.async_copy(table.at[sidx.at[k]], rows.at[b], gsem.at[b])

        def g_wait(k, b):
            pltpu.make_async_copy(
                table.at[sidx.at[k]], rows.at[b], gsem.at[b]).wait()

        def s_start(k, b):
            pltpu.async_copy(
                rows.at[b], acc.at[didx.at[k]], ssem.at[b], add=True)

        def s_wait(k, b):
            pltpu.make_async_copy(
                rows.at[b], acc.at[didx.at[k]], ssem.at[b]).wait()

        for b in range(NBUF):
            g_start(b, b)

        def step(t, _):
            for b in range(NBUF):
                k = t * NBUF + b
                g_wait(k, b)
                s_start(k, b)
            for b in range(NBUF):
                k = t * NBUF + b
                s_wait(k, b)
                g_start(k + NBUF, b)
            return 0
        lax.fori_loop(0, NCH // NBUF - 1, step, 0)

        t_last = NCH - NBUF
        for b in range(NBUF):
            g_wait(t_last + b, b)
            s_start(t_last + b, b)
        for b in range(NBUF):
            s_wait(t_last + b, b)

        plsc.subcore_barrier()
        for kk in range(NPT // CH):
            r0 = s * NPT + kk * CH
            pltpu.sync_copy(acc.at[pl.ds(r0, CH)], rows.at[0])
            pltpu.sync_copy(rows.at[0], out.at[pl.ds(c * NPAD + r0, CH)])

    return pl.kernel(
        body,
        out_type=jax.ShapeDtypeStruct((2 * NPAD, D), jnp.float32),
        mesh=_mesh(),
        compiler_params=_SC_PARAMS,
        scratch_types=[
            pltpu.VMEM_SHARED((NPAD, D), jnp.float32),
            pltpu.VMEM((NCH, CH), jnp.int32),
            pltpu.VMEM((NCH, CH), jnp.int32),
            pltpu.VMEM((NBUF, CH, D), jnp.float32),
            pltpu.SemaphoreType.DMA((NBUF,)),
            pltpu.SemaphoreType.DMA((NBUF,)),
        ],
    )


def _h0_body(x_ref, dgo_ref, o_ref):
    x = x_ref[...]
    dego = jnp.sum(dgo_ref[...], axis=0)
    norm_out = jnp.where(dego > 0, lax.rsqrt(dego), 0.0)
    xn = jnp.sqrt(jnp.sum(x * x, axis=-1))
    scale = norm_out / jnp.maximum(xn, EPS)
    o_ref[...] = x * scale[:, None]


def _h0_call(xp, degp):
    return pl.pallas_call(
        _h0_body,
        grid=(NBLK,),
        in_specs=[
            pl.BlockSpec((RB, DP), lambda i: (i, 0)),
            pl.BlockSpec((2, RB), lambda i: (0, i)),
        ],
        out_specs=pl.BlockSpec((RB, DP), lambda i: (i, 0)),
        out_shape=jax.ShapeDtypeStruct((NPAD, DP), jnp.float32),
    )(xp, degp)


def _l1_body(a0_ref, a1_ref, dgo_ref, dgi_ref, w1_ref, b1_ref, w2_ref, o_ref):
    dego = jnp.sum(dgo_ref[...], axis=0)
    degi = jnp.sum(dgi_ref[...], axis=0)
    no = jnp.where(dego > 0, lax.rsqrt(dego), 0.0)
    ni = jnp.where(degi > 0, lax.rsqrt(degi), 0.0)
    agg = (a0_ref[...] + a1_ref[...]) * ni[:, None]
    h = jnp.dot(agg, w1_ref[...], preferred_element_type=jnp.float32)
    h = h + b1_ref[...]
    hn = jnp.sqrt(jnp.sum(h * h, axis=-1, keepdims=True))
    h = h / jnp.maximum(hn, EPS)
    h = jnp.maximum(h, 0.0)
    hn = jnp.sqrt(jnp.sum(h * h, axis=-1, keepdims=True))
    h = h / jnp.maximum(hn, EPS)
    h = h * no[:, None]
    o_ref[...] = jnp.dot(h, w2_ref[...], preferred_element_type=jnp.float32)


def _l1_call(agg1, degp, w1p, b1r, w2):
    return pl.pallas_call(
        _l1_body,
        grid=(NBLK,),
        in_specs=[
            pl.BlockSpec((RB, DP), lambda i: (i, 0)),
            pl.BlockSpec((RB, DP), lambda i: (i + NBLK, 0)),
            pl.BlockSpec((2, RB), lambda i: (0, i)),
            pl.BlockSpec((2, RB), lambda i: (0, i + NBLK)),
            pl.BlockSpec((DP, HID), lambda i: (0, 0)),
            pl.BlockSpec((1, HID), lambda i: (0, 0)),
            pl.BlockSpec((HID, OUT2), lambda i: (0, 0)),
        ],
        out_specs=pl.BlockSpec((RB, OUT2), lambda i: (i, 0)),
        out_shape=jax.ShapeDtypeStruct((NPAD, OUT2), jnp.float32),
    )(agg1, agg1, degp, degp, w1p, b1r, w2)


def _l2_body(q0_ref, q1_ref, dgi_ref, b2_ref, wc_ref, bc_ref, o_ref, acc_ref):
    i = pl.program_id(0)
    degi = jnp.sum(dgi_ref[...], axis=0)
    ni = jnp.where(degi > 0, lax.rsqrt(degi), 0.0)
    h = (q0_ref[...] + q1_ref[...]) * ni[:, None] + b2_ref[...]
    hn = jnp.sqrt(jnp.sum(h * h, axis=-1, keepdims=True))
    h = h / jnp.maximum(hn, EPS)
    h = jnp.maximum(h, 0.0)
    hn = jnp.sqrt(jnp.sum(h * h, axis=-1, keepdims=True))
    h = h / jnp.maximum(hn, EPS)
    rid = lax.broadcasted_iota(jnp.int32, (RB, 1), 0) + i * RB
    h = jnp.where(rid < N, h, 0.0)
    part = jnp.sum(h, axis=0, keepdims=True)

    @pl.when(i == 0)
    def _():
        acc_ref[...] = part

    @pl.when(i > 0)
    def _():
        acc_ref[...] = acc_ref[...] + part

    @pl.when(i == NBLK - 1)
    def _():
        hg = acc_ref[...] / N
        hgn = jnp.sqrt(jnp.sum(hg * hg))
        hg = hg / jnp.maximum(hgn, EPS)
        o_ref[...] = (
            jnp.dot(hg, wc_ref[...], preferred_element_type=jnp.float32)
            + bc_ref[...])


def _l2_call(agg2, degp, b2r, wc, bcr):
    return pl.pallas_call(
        _l2_body,
        grid=(NBLK,),
        in_specs=[
            pl.BlockSpec((RB, OUT2), lambda i: (i, 0)),
            pl.BlockSpec((RB, OUT2), lambda i: (i + NBLK, 0)),
            pl.BlockSpec((2, RB), lambda i: (0, i + NBLK)),
            pl.BlockSpec((1, OUT2), lambda i: (0, 0)),
            pl.BlockSpec((OUT2, NCLS), lambda i: (0, 0)),
            pl.BlockSpec((1, NCLS), lambda i: (0, 0)),
        ],
        out_specs=pl.BlockSpec((1, NCLS), lambda i: (0, 0)),
        out_shape=jax.ShapeDtypeStruct((1, NCLS), jnp.float32),
        scratch_shapes=[pltpu.VMEM((1, OUT2), jnp.float32)],
        compiler_params=pltpu.CompilerParams(
            dimension_semantics=("arbitrary",)),
    )(agg2, agg2, degp, b2r, wc, bcr)


@jax.jit
def kernel(node_embed_weight, edge_index, W1, b1, W2, b2, Wc, bc):
    f32 = jnp.float32
    src = edge_index[0].astype(jnp.int32)
    dst = edge_index[1].astype(jnp.int32)
    npe = EPAD - E
    pad_idx = N + (jnp.arange(npe, dtype=jnp.int32) % (NPAD - N))
    srcp = jnp.concatenate([src, pad_idx]).reshape(EPAD // CH, CH)
    dstp = jnp.concatenate([dst, pad_idx]).reshape(EPAD // CH, CH)
    dstp2 = dstp + NPAD

    xp = jnp.zeros((NPAD, DP), f32).at[:N, :IN_DIM].set(
        node_embed_weight.astype(f32))
    w1p = jnp.zeros((DP, HID), f32).at[:IN_DIM].set(W1.astype(f32))

    degp = _deg_kernel()(srcp, dstp2).reshape(2, 2 * NPAD)
    h0s = _h0_call(xp, degp)
    agg1 = _make_agg(DP)(h0s, srcp, dstp)
    p = _l1_call(agg1, degp, w1p, b1.reshape(1, HID), W2)
    agg2 = _make_agg(OUT2)(p, srcp, dstp)
    return _l2_call(agg2, degp, b2.reshape(1, OUT2), Wc, bc.reshape(1, NCLS))


# speedup vs baseline: 18.8301x; 1.2589x over previous
import functools

import jax
import jax.numpy as jnp
from jax import lax
from jax.experimental import pallas as pl
from jax.experimental.pallas import tpu as pltpu
from jax.experimental.pallas import tpu_sc as plsc

N = 10000
E = 160000
IN_DIM = 60
DP = 64
HID = 512
OUT2 = 32
NCLS = 16

NC, NS = 2, 16
NW = NC * NS
NPAD = 10240
NPT = NPAD // NS
CH = 128
EW = 5120
EPAD = NW * EW
NCH = EW // CH
RB = 1024
NBLK = NPAD // RB
EPS = 1e-12

@functools.cache
def _mesh():
    return plsc.VectorSubcoreMesh(
        core_axis_name="c", subcore_axis_name="s",
        num_cores=NC, num_subcores=NS)


def _deg_body(srcp2, dstp2, out, acc, sidx, didx, ones, zbuf, dsem):
    c = lax.axis_index("c")
    s = lax.axis_index("s")
    gid = c * NS + s
    z16 = jnp.zeros((16,), jnp.float32)
    o16 = jnp.ones((16,), jnp.float32)
    zn = (2 * NPAD) // NS

    def zb(i, _):
        zbuf[pl.ds(i * 16, 16)] = z16
        return 0
    lax.fori_loop(0, zn // 16, zb, 0)

    def ob(i, _):
        ones[pl.ds(i * 16, 16)] = o16
        return 0
    lax.fori_loop(0, CH // 16, ob, 0)

    pltpu.sync_copy(zbuf, acc.at[pl.ds(s * zn, zn)])
    plsc.subcore_barrier()

    pltpu.sync_copy(srcp2.at[pl.ds(gid * NCH, NCH)], sidx)
    pltpu.sync_copy(dstp2.at[pl.ds(gid * NCH, NCH)], didx)

    FD = 4

    def step(t, _):
        for j in range(FD):
            k = t * FD + j
            pltpu.async_copy(ones, acc.at[sidx.at[k]], dsem, add=True)
            pltpu.async_copy(ones, acc.at[didx.at[k]], dsem, add=True)
        for j in range(FD):
            k = t * FD + j
            pltpu.make_async_copy(ones, acc.at[sidx.at[k]], dsem).wait()
            pltpu.make_async_copy(ones, acc.at[didx.at[k]], dsem).wait()
        return 0
    lax.fori_loop(0, NCH // FD, step, 0)

    plsc.subcore_barrier()
    pltpu.sync_copy(acc.at[pl.ds(s * zn, zn)], zbuf)
    pltpu.sync_copy(zbuf, out.at[pl.ds(c * 2 * NPAD + s * zn, zn)])


_SC_PARAMS = pltpu.CompilerParams(use_tc_tiling_on_sc=False)


@functools.cache
def _deg_kernel():
    return pl.kernel(
        _deg_body,
        out_type=jax.ShapeDtypeStruct((2 * 2 * NPAD,), jnp.float32),
        mesh=_mesh(),
        compiler_params=_SC_PARAMS,
        scratch_types=[
            pltpu.VMEM_SHARED((2 * NPAD,), jnp.float32),
            pltpu.VMEM((NCH, CH), jnp.int32),
            pltpu.VMEM((NCH, CH), jnp.int32),
            pltpu.VMEM((CH,), jnp.float32),
            pltpu.VMEM(((2 * NPAD) // NS,), jnp.float32),
            pltpu.SemaphoreType.DMA,
        ],
    )


NBUF = 8


@functools.cache
def _make_agg(D):
    def body(table, srcp2, dstp2, out, acc, sidx, didx, rows, gsem, ssem):
        c = lax.axis_index("c")
        s = lax.axis_index("s")
        gid = c * NS + s
        z16 = jnp.zeros((16,), jnp.float32)

        def zr(i, _):
            def zc(j, _):
                rows[0, i, pl.ds(j * 16, 16)] = z16
                return 0
            return lax.fori_loop(0, D // 16, zc, 0)
        lax.fori_loop(0, CH, zr, 0)

        for kk in range(NPT // CH):
            pltpu.sync_copy(rows.at[0], acc.at[pl.ds(s * NPT + kk * CH, CH)])
        plsc.subcore_barrier()

        pltpu.sync_copy(srcp2.at[pl.ds(gid * NCH, NCH)], sidx)
        pltpu.sync_copy(dstp2.at[pl.ds(gid * NCH, NCH)], didx)

        def g_start(k, b):
            pltpu.async_copy(table.at[sidx.at[k]], rows.at[b], gsem.at[b])

        def g_wait(k, b):
            pltpu.make_async_copy(
                table.at[sidx.at[k]], rows.at[b], gsem.at[b]).wait()

        def s_start(k, b):
            pltpu.async_copy(
                rows.at[b], acc.at[didx.at[k]], ssem.at[b], add=True)

        def s_wait(k, b):
            pltpu.make_async_copy(
                rows.at[b], acc.at[didx.at[k]], ssem.at[b]).wait()

        for b in range(NBUF):
            g_start(b, b)

        def step(t, _):
            for b in range(NBUF):
                k = t * NBUF + b
                g_wait(k, b)
                s_start(k, b)
            for b in range(NBUF):
                k = t * NBUF + b
                s_wait(k, b)
                g_start(k + NBUF, b)
            return 0
        lax.fori_loop(0, NCH // NBUF - 1, step, 0)

        t_last = NCH - NBUF
        for b in range(NBUF):
            g_wait(t_last + b, b)
            s_start(t_last + b, b)
        for b in range(NBUF):
            s_wait(t_last + b, b)

        plsc.subcore_barrier()
        for kk in range(NPT // CH):
            r0 = s * NPT + kk * CH
            pltpu.sync_copy(acc.at[pl.ds(r0, CH)], rows.at[0])
            pltpu.sync_copy(rows.at[0], out.at[pl.ds(c * NPAD + r0, CH)])

    return pl.kernel(
        body,
        out_type=jax.ShapeDtypeStruct((2 * NPAD, D), jnp.float32),
        mesh=_mesh(),
        compiler_params=_SC_PARAMS,
        scratch_types=[
            pltpu.VMEM_SHARED((NPAD, D), jnp.float32),
            pltpu.VMEM((NCH, CH), jnp.int32),
            pltpu.VMEM((NCH, CH), jnp.int32),
            pltpu.VMEM((NBUF, CH, D), jnp.float32),
            pltpu.SemaphoreType.DMA((NBUF,)),
            pltpu.SemaphoreType.DMA((NBUF,)),
        ],
    )


def _h0_body(x_ref, dgo_ref, o_ref):
    x = x_ref[...]
    dego = jnp.sum(dgo_ref[...], axis=0)
    norm_out = jnp.where(dego > 0, lax.rsqrt(dego), 0.0)
    xn = jnp.sqrt(jnp.sum(x * x, axis=-1))
    scale = norm_out / jnp.maximum(xn, EPS)
    o_ref[...] = x * scale[:, None]


def _h0_call(xp, degp):
    return pl.pallas_call(
        _h0_body,
        grid=(NBLK,),
        in_specs=[
            pl.BlockSpec((RB, DP), lambda i: (i, 0)),
            pl.BlockSpec((2, RB), lambda i: (0, i)),
        ],
        out_specs=pl.BlockSpec((RB, DP), lambda i: (i, 0)),
        out_shape=jax.ShapeDtypeStruct((NPAD, DP), jnp.float32),
    )(xp, degp)


def _l1_body(a0_ref, a1_ref, dgo_ref, dgi_ref, w1_ref, b1_ref, w2_ref, o_ref):
    dego = jnp.sum(dgo_ref[...], axis=0)
    degi = jnp.sum(dgi_ref[...], axis=0)
    no = jnp.where(dego > 0, lax.rsqrt(dego), 0.0)
    ni = jnp.where(degi > 0, lax.rsqrt(degi), 0.0)
    agg = (a0_ref[...] + a1_ref[...]) * ni[:, None]
    h = jnp.dot(agg, w1_ref[...], preferred_element_type=jnp.float32)
    h = h + b1_ref[...]
    hn = jnp.sqrt(jnp.sum(h * h, axis=-1, keepdims=True))
    h = h / jnp.maximum(hn, EPS)
    h = jnp.maximum(h, 0.0)
    hn = jnp.sqrt(jnp.sum(h * h, axis=-1, keepdims=True))
    h = h / jnp.maximum(hn, EPS)
    h = h * no[:, None]
    o_ref[...] = jnp.dot(h, w2_ref[...], preferred_element_type=jnp.float32)


def _l1_call(agg1, degp, w1p, b1r, w2):
    return pl.pallas_call(
        _l1_body,
        grid=(NBLK,),
        in_specs=[
            pl.BlockSpec((RB, DP), lambda i: (i, 0)),
            pl.BlockSpec((RB, DP), lambda i: (i + NBLK, 0)),
            pl.BlockSpec((2, RB), lambda i: (0, i)),
            pl.BlockSpec((2, RB), lambda i: (0, i + NBLK)),
            pl.BlockSpec((DP, HID), lambda i: (0, 0)),
            pl.BlockSpec((1, HID), lambda i: (0, 0)),
            pl.BlockSpec((HID, OUT2), lambda i: (0, 0)),
        ],
        out_specs=pl.BlockSpec((RB, OUT2), lambda i: (i, 0)),
        out_shape=jax.ShapeDtypeStruct((NPAD, OUT2), jnp.float32),
    )(agg1, agg1, degp, degp, w1p, b1r, w2)


def _l2_body(q0_ref, q1_ref, dgi_ref, b2_ref, wc_ref, bc_ref, o_ref, acc_ref):
    i = pl.program_id(0)
    degi = jnp.sum(dgi_ref[...], axis=0)
    ni = jnp.where(degi > 0, lax.rsqrt(degi), 0.0)
    h = (q0_ref[...] + q1_ref[...]) * ni[:, None] + b2_ref[...]
    hn = jnp.sqrt(jnp.sum(h * h, axis=-1, keepdims=True))
    h = h / jnp.maximum(hn, EPS)
    h = jnp.maximum(h, 0.0)
    hn = jnp.sqrt(jnp.sum(h * h, axis=-1, keepdims=True))
    h = h / jnp.maximum(hn, EPS)
    rid = lax.broadcasted_iota(jnp.int32, (RB, 1), 0) + i * RB
    h = jnp.where(rid < N, h, 0.0)
    part = jnp.sum(h, axis=0, keepdims=True)

    @pl.when(i == 0)
    def _():
        acc_ref[...] = part

    @pl.when(i > 0)
    def _():
        acc_ref[...] = acc_ref[...] + part

    @pl.when(i == NBLK - 1)
    def _():
        hg = acc_ref[...] / N
        hgn = jnp.sqrt(jnp.sum(hg * hg))
        hg = hg / jnp.maximum(hgn, EPS)
        o_ref[...] = (
            jnp.dot(hg, wc_ref[...], preferred_element_type=jnp.float32)
            + bc_ref[...])


def _l2_call(agg2, degp, b2r, wc, bcr):
    return pl.pallas_call(
        _l2_body,
        grid=(NBLK,),
        in_specs=[
            pl.BlockSpec((RB, OUT2), lambda i: (i, 0)),
            pl.BlockSpec((RB, OUT2), lambda i: (i + NBLK, 0)),
            pl.BlockSpec((2, RB), lambda i: (0, i + NBLK)),
            pl.BlockSpec((1, OUT2), lambda i: (0, 0)),
            pl.BlockSpec((OUT2, NCLS), lambda i: (0, 0)),
            pl.BlockSpec((1, NCLS), lambda i: (0, 0)),
        ],
        out_specs=pl.BlockSpec((1, NCLS), lambda i: (0, 0)),
        out_shape=jax.ShapeDtypeStruct((1, NCLS), jnp.float32),
        scratch_shapes=[pltpu.VMEM((1, OUT2), jnp.float32)],
        compiler_params=pltpu.CompilerParams(
            dimension_semantics=("arbitrary",)),
    )(agg2, agg2, degp, b2r, wc, bcr)


@jax.jit
def kernel(node_embed_weight, edge_index, W1, b1, W2, b2, Wc, bc):
    f32 = jnp.float32
    src = edge_index[0].astype(jnp.int32)
    dst = edge_index[1].astype(jnp.int32)
    npe = EPAD - E
    pad_idx = N + (jnp.arange(npe, dtype=jnp.int32) % (NPAD - N))
    srcp = jnp.concatenate([src, pad_idx]).reshape(EPAD // CH, CH)
    dstp = jnp.concatenate([dst, pad_idx]).reshape(EPAD // CH, CH)
    dstp2 = dstp + NPAD

    xp = jnp.zeros((NPAD, DP), f32).at[:N, :IN_DIM].set(
        node_embed_weight.astype(f32))
    w1p = jnp.zeros((DP, HID), f32).at[:IN_DIM].set(W1.astype(f32))

    degp = _deg_kernel()(srcp, dstp2).reshape(2, 2 * NPAD)
    h0s = _h0_call(xp, degp)
    agg1 = _make_agg(DP)(h0s, srcp, dstp)
    p = _l1_call(agg1, degp, w1p, b1.reshape(1, HID), W2)
    agg2 = _make_agg(OUT2)(p, srcp, dstp)
    return _l2_call(agg2, degp, b2.reshape(1, OUT2), Wc, bc.reshape(1, NCLS))


# speedup vs baseline: 18.8795x; 1.0026x over previous
import functools

import jax
import jax.numpy as jnp
from jax import lax
from jax.experimental import pallas as pl
from jax.experimental.pallas import tpu as pltpu
from jax.experimental.pallas import tpu_sc as plsc

N = 10000
E = 160000
IN_DIM = 60
DP = 64
HID = 512
OUT2 = 32
NCLS = 16

NC, NS = 2, 16
NW = NC * NS
NPAD = 10240
NPT = NPAD // NS
CH = 128
EW = 5120
EPAD = NW * EW
NCH = EW // CH
RB = 1024
NBLK = NPAD // RB
EPS = 1e-12

@functools.cache
def _mesh():
    return plsc.VectorSubcoreMesh(
        core_axis_name="c", subcore_axis_name="s",
        num_cores=NC, num_subcores=NS)


def _deg_body(srcp2, dstp2, out, acco, acci, sidx, didx, ones, zbuf, dsem):
    c = lax.axis_index("c")
    s = lax.axis_index("s")
    gid = c * NS + s
    z16 = jnp.zeros((16,), jnp.float32)
    o16 = jnp.ones((16,), jnp.float32)
    zn = NPAD // NS

    def zb(i, _):
        zbuf[pl.ds(i * 16, 16)] = z16
        return 0
    lax.fori_loop(0, zn // 16, zb, 0)

    def ob(i, _):
        ones[pl.ds(i * 16, 16)] = o16
        return 0
    lax.fori_loop(0, CH // 16, ob, 0)

    pltpu.sync_copy(zbuf, acco.at[pl.ds(s * zn, zn)])
    pltpu.sync_copy(zbuf, acci.at[pl.ds(s * zn, zn)])
    plsc.subcore_barrier()

    pltpu.sync_copy(srcp2.at[pl.ds(gid * NCH, NCH)], sidx)
    pltpu.sync_copy(dstp2.at[pl.ds(gid * NCH, NCH)], didx)

    FD = 4

    def step(t, _):
        for j in range(FD):
            k = t * FD + j
            pltpu.async_copy(ones, acco.at[sidx.at[k]], dsem, add=True)
            pltpu.async_copy(ones, acci.at[didx.at[k]], dsem, add=True)
        for j in range(FD):
            k = t * FD + j
            pltpu.make_async_copy(ones, acco.at[sidx.at[k]], dsem).wait()
            pltpu.make_async_copy(ones, acci.at[didx.at[k]], dsem).wait()
        return 0
    lax.fori_loop(0, NCH // FD, step, 0)

    plsc.subcore_barrier()
    pltpu.sync_copy(acco.at[pl.ds(s * zn, zn)], zbuf)
    pltpu.sync_copy(zbuf, out.at[pl.ds(c * 2 * NPAD + s * zn, zn)])
    pltpu.sync_copy(acci.at[pl.ds(s * zn, zn)], zbuf)
    pltpu.sync_copy(zbuf, out.at[pl.ds(c * 2 * NPAD + NPAD + s * zn, zn)])


_SC_PARAMS = pltpu.CompilerParams(use_tc_tiling_on_sc=False)


@functools.cache
def _deg_kernel():
    return pl.kernel(
        _deg_body,
        out_type=jax.ShapeDtypeStruct((2 * 2 * NPAD,), jnp.float32),
        mesh=_mesh(),
        compiler_params=_SC_PARAMS,
        scratch_types=[
            pltpu.VMEM_SHARED((NPAD,), jnp.float32),
            pltpu.VMEM_SHARED((NPAD,), jnp.float32),
            pltpu.VMEM((NCH, CH), jnp.int32),
            pltpu.VMEM((NCH, CH), jnp.int32),
            pltpu.VMEM((CH,), jnp.float32),
            pltpu.VMEM((NPAD // NS,), jnp.float32),
            pltpu.SemaphoreType.DMA,
        ],
    )


NBUF = 8


@functools.cache
def _make_agg(D):
    def body(table, srcp2, dstp2, out, acc, sidx, didx, rows, gsem, ssem):
        c = lax.axis_index("c")
        s = lax.axis_index("s")
        gid = c * NS + s
        z16 = jnp.zeros((16,), jnp.float32)

        def zr(i, _):
            def zc(j, _):
                rows[0, i, pl.ds(j * 16, 16)] = z16
                return 0
            return lax.fori_loop(0, D // 16, zc, 0)
        lax.fori_loop(0, CH, zr, 0)

        for kk in range(NPT // CH):
            pltpu.sync_copy(rows.at[0], acc.at[pl.ds(s * NPT + kk * CH, CH)])
        plsc.subcore_barrier()

        pltpu.sync_copy(srcp2.at[pl.ds(gid * NCH, NCH)], sidx)
        pltpu.sync_copy(dstp2.at[pl.ds(gid * NCH, NCH)], didx)

        def g_start(k, b):
            pltpu.async_copy(table.at[sidx.at[k]], rows.at[b], gsem.at[b])

        def g_wait(k, b):
            pltpu.make_async_copy(
                table.at[sidx.at[k]], rows.at[b], gsem.at[b]).wait()

        def s_start(k, b):
            pltpu.async_copy(
                rows.at[b], acc.at[didx.at[k]], ssem.at[b], add=True)

        def s_wait(k, b):
            pltpu.make_async_copy(
                rows.at[b], acc.at[didx.at[k]], ssem.at[b]).wait()

        for b in range(NBUF):
            g_start(b, b)

        def step(t, _):
            for b in range(NBUF):
                k = t * NBUF + b
                g_wait(k, b)
                s_start(k, b)
            for b in range(NBUF):
                k = t * NBUF + b
                s_wait(k, b)
                g_start(k + NBUF, b)
            return 0
        lax.fori_loop(0, NCH // NBUF - 1, step, 0)

        t_last = NCH - NBUF
        for b in range(NBUF):
            g_wait(t_last + b, b)
            s_start(t_last + b, b)
        for b in range(NBUF):
            s_wait(t_last + b, b)

        plsc.subcore_barrier()
        for kk in range(NPT // CH):
            r0 = s * NPT + kk * CH
            pltpu.sync_copy(acc.at[pl.ds(r0, CH)], rows.at[0])
            pltpu.sync_copy(rows.at[0], out.at[pl.ds(c * NPAD + r0, CH)])

    return pl.kernel(
        body,
        out_type=jax.ShapeDtypeStruct((2 * NPAD, D), jnp.float32),
        mesh=_mesh(),
        compiler_params=_SC_PARAMS,
        scratch_types=[
            pltpu.VMEM_SHARED((NPAD, D), jnp.float32),
            pltpu.VMEM((NCH, CH), jnp.int32),
            pltpu.VMEM((NCH, CH), jnp.int32),
            pltpu.VMEM((NBUF, CH, D), jnp.float32),
            pltpu.SemaphoreType.DMA((NBUF,)),
            pltpu.SemaphoreType.DMA((NBUF,)),
        ],
    )


def _h0_body(x_ref, dgo_ref, o_ref):
    x = x_ref[...]
    dego = jnp.sum(dgo_ref[...], axis=0)
    norm_out = jnp.where(dego > 0, lax.rsqrt(dego), 0.0)
    xn = jnp.sqrt(jnp.sum(x * x, axis=-1))
    scale = norm_out / jnp.maximum(xn, EPS)
    o_ref[...] = x * scale[:, None]


def _h0_call(xp, degp):
    return pl.pallas_call(
        _h0_body,
        grid=(NBLK,),
        in_specs=[
            pl.BlockSpec((RB, DP), lambda i: (i, 0)),
            pl.BlockSpec((2, RB), lambda i: (0, i)),
        ],
        out_specs=pl.BlockSpec((RB, DP), lambda i: (i, 0)),
        out_shape=jax.ShapeDtypeStruct((NPAD, DP), jnp.float32),
    )(xp, degp)


def _l1_body(a0_ref, a1_ref, dgo_ref, dgi_ref, w1_ref, b1_ref, w2_ref, o_ref):
    dego = jnp.sum(dgo_ref[...], axis=0)
    degi = jnp.sum(dgi_ref[...], axis=0)
    no = jnp.where(dego > 0, lax.rsqrt(dego), 0.0)
    ni = jnp.where(degi > 0, lax.rsqrt(degi), 0.0)
    agg = (a0_ref[...] + a1_ref[...]) * ni[:, None]
    h = jnp.dot(agg.astype(jnp.bfloat16), w1_ref[...].astype(jnp.bfloat16),
                preferred_element_type=jnp.float32)
    h = h + b1_ref[...]
    hn = jnp.sqrt(jnp.sum(h * h, axis=-1, keepdims=True))
    h = h / jnp.maximum(hn, EPS)
    h = jnp.maximum(h, 0.0)
    hn = jnp.sqrt(jnp.sum(h * h, axis=-1, keepdims=True))
    h = h / jnp.maximum(hn, EPS)
    h = h * no[:, None]
    o_ref[...] = jnp.dot(h.astype(jnp.bfloat16),
                         w2_ref[...].astype(jnp.bfloat16),
                         preferred_element_type=jnp.float32)


def _l1_call(agg1, degp, w1p, b1r, w2):
    return pl.pallas_call(
        _l1_body,
        grid=(NBLK,),
        in_specs=[
            pl.BlockSpec((RB, DP), lambda i: (i, 0)),
            pl.BlockSpec((RB, DP), lambda i: (i + NBLK, 0)),
            pl.BlockSpec((2, RB), lambda i: (0, i)),
            pl.BlockSpec((2, RB), lambda i: (0, i + NBLK)),
            pl.BlockSpec((DP, HID), lambda i: (0, 0)),
            pl.BlockSpec((1, HID), lambda i: (0, 0)),
            pl.BlockSpec((HID, OUT2), lambda i: (0, 0)),
        ],
        out_specs=pl.BlockSpec((RB, OUT2), lambda i: (i, 0)),
        out_shape=jax.ShapeDtypeStruct((NPAD, OUT2), jnp.float32),
    )(agg1, agg1, degp, degp, w1p, b1r, w2)


def _l2_body(q0_ref, q1_ref, dgi_ref, b2_ref, wc_ref, bc_ref, o_ref, acc_ref):
    i = pl.program_id(0)
    degi = jnp.sum(dgi_ref[...], axis=0)
    ni = jnp.where(degi > 0, lax.rsqrt(degi), 0.0)
    h = (q0_ref[...] + q1_ref[...]) * ni[:, None] + b2_ref[...]
    hn = jnp.sqrt(jnp.sum(h * h, axis=-1, keepdims=True))
    h = h / jnp.maximum(hn, EPS)
    h = jnp.maximum(h, 0.0)
    hn = jnp.sqrt(jnp.sum(h * h, axis=-1, keepdims=True))
    h = h / jnp.maximum(hn, EPS)
    rid = lax.broadcasted_iota(jnp.int32, (RB, 1), 0) + i * RB
    h = jnp.where(rid < N, h, 0.0)
    part = jnp.sum(h, axis=0, keepdims=True)

    @pl.when(i == 0)
    def _():
        acc_ref[...] = part

    @pl.when(i > 0)
    def _():
        acc_ref[...] = acc_ref[...] + part

    @pl.when(i == NBLK - 1)
    def _():
        hg = acc_ref[...] / N
        hgn = jnp.sqrt(jnp.sum(hg * hg))
        hg = hg / jnp.maximum(hgn, EPS)
        o_ref[...] = (
            jnp.dot(hg, wc_ref[...], preferred_element_type=jnp.float32)
            + bc_ref[...])


def _l2_call(agg2, degp, b2r, wc, bcr):
    return pl.pallas_call(
        _l2_body,
        grid=(NBLK,),
        in_specs=[
            pl.BlockSpec((RB, OUT2), lambda i: (i, 0)),
            pl.BlockSpec((RB, OUT2), lambda i: (i + NBLK, 0)),
            pl.BlockSpec((2, RB), lambda i: (0, i + NBLK)),
            pl.BlockSpec((1, OUT2), lambda i: (0, 0)),
            pl.BlockSpec((OUT2, NCLS), lambda i: (0, 0)),
            pl.BlockSpec((1, NCLS), lambda i: (0, 0)),
        ],
        out_specs=pl.BlockSpec((1, NCLS), lambda i: (0, 0)),
        out_shape=jax.ShapeDtypeStruct((1, NCLS), jnp.float32),
        scratch_shapes=[pltpu.VMEM((1, OUT2), jnp.float32)],
        compiler_params=pltpu.CompilerParams(
            dimension_semantics=("arbitrary",)),
    )(agg2, agg2, degp, b2r, wc, bcr)


@jax.jit
def kernel(node_embed_weight, edge_index, W1, b1, W2, b2, Wc, bc):
    f32 = jnp.float32
    src = edge_index[0].astype(jnp.int32)
    dst = edge_index[1].astype(jnp.int32)
    npe = EPAD - E
    pad_idx = N + (jnp.arange(npe, dtype=jnp.int32) % (NPAD - N))
    srcp = jnp.concatenate([src, pad_idx]).reshape(EPAD // CH, CH)
    dstp = jnp.concatenate([dst, pad_idx]).reshape(EPAD // CH, CH)

    xp = jnp.zeros((NPAD, DP), f32).at[:N, :IN_DIM].set(
        node_embed_weight.astype(f32))
    w1p = jnp.zeros((DP, HID), f32).at[:IN_DIM].set(W1.astype(f32))

    degp = _deg_kernel()(srcp, dstp).reshape(2, 2 * NPAD)
    h0s = _h0_call(xp, degp)
    agg1 = _make_agg(DP)(h0s, srcp, dstp)
    p = _l1_call(agg1, degp, w1p, b1.reshape(1, HID), W2)
    agg2 = _make_agg(OUT2)(p, srcp, dstp)
    return _l2_call(agg2, degp, b2.reshape(1, OUT2), Wc, bc.reshape(1, NCLS))


# speedup vs baseline: 20.1736x; 1.0685x over previous
import functools

import jax
import jax.numpy as jnp
from jax import lax
from jax.experimental import pallas as pl
from jax.experimental.pallas import tpu as pltpu
from jax.experimental.pallas import tpu_sc as plsc

N = 10000
E = 160000
IN_DIM = 60
DP = 64
HID = 512
OUT2 = 32
NCLS = 16

NC, NS = 2, 16
NW = NC * NS
NPAD = 10240
NPT = NPAD // NS
CH = 128
EW = 5120
EPAD = NW * EW
NCH = EW // CH
RB = 1024
NBLK = NPAD // RB
EPS = 1e-12

@functools.cache
def _mesh():
    return plsc.VectorSubcoreMesh(
        core_axis_name="c", subcore_axis_name="s",
        num_cores=NC, num_subcores=NS)


def _deg_body(srcp2, dstp2, out, acco, acci, sidx, didx, ones, zbuf, dsem):
    c = lax.axis_index("c")
    s = lax.axis_index("s")
    gid = c * NS + s
    z16 = jnp.zeros((16,), jnp.float32)
    o16 = jnp.ones((16,), jnp.float32)
    zn = NPAD // NS

    def zb(i, _):
        zbuf[pl.ds(i * 16, 16)] = z16
        return 0
    lax.fori_loop(0, zn // 16, zb, 0)

    def ob(i, _):
        ones[pl.ds(i * 16, 16)] = o16
        return 0
    lax.fori_loop(0, CH // 16, ob, 0)

    pltpu.sync_copy(zbuf, acco.at[pl.ds(s * zn, zn)])
    pltpu.sync_copy(zbuf, acci.at[pl.ds(s * zn, zn)])
    plsc.subcore_barrier()

    pltpu.sync_copy(srcp2.at[pl.ds(gid * NCH, NCH)], sidx)
    pltpu.sync_copy(dstp2.at[pl.ds(gid * NCH, NCH)], didx)

    FD = 4

    def step(t, _):
        for j in range(FD):
            k = t * FD + j
            pltpu.async_copy(ones, acco.at[sidx.at[k]], dsem, add=True)
            pltpu.async_copy(ones, acci.at[didx.at[k]], dsem, add=True)
        for j in range(FD):
            k = t * FD + j
            pltpu.make_async_copy(ones, acco.at[sidx.at[k]], dsem).wait()
            pltpu.make_async_copy(ones, acci.at[didx.at[k]], dsem).wait()
        return 0
    lax.fori_loop(0, NCH // FD, step, 0)

    plsc.subcore_barrier()
    pltpu.sync_copy(acco.at[pl.ds(s * zn, zn)], zbuf)
    pltpu.sync_copy(zbuf, out.at[pl.ds(c * 2 * NPAD + s * zn, zn)])
    pltpu.sync_copy(acci.at[pl.ds(s * zn, zn)], zbuf)
    pltpu.sync_copy(zbuf, out.at[pl.ds(c * 2 * NPAD + NPAD + s * zn, zn)])


_SC_PARAMS = pltpu.CompilerParams(use_tc_tiling_on_sc=False)


@functools.cache
def _deg_kernel():
    return pl.kernel(
        _deg_body,
        out_type=jax.ShapeDtypeStruct((2 * 2 * NPAD,), jnp.float32),
        mesh=_mesh(),
        compiler_params=_SC_PARAMS,
        scratch_types=[
            pltpu.VMEM_SHARED((NPAD,), jnp.float32),
            pltpu.VMEM_SHARED((NPAD,), jnp.float32),
            pltpu.VMEM((NCH, CH), jnp.int32),
            pltpu.VMEM((NCH, CH), jnp.int32),
            pltpu.VMEM((CH,), jnp.float32),
            pltpu.VMEM((NPAD // NS,), jnp.float32),
            pltpu.SemaphoreType.DMA,
        ],
    )


NBUF = 8


@functools.cache
def _make_agg(D):
    def body(table, srcp2, dstp2, out, acc, sidx, didx, rows, gsem, ssem):
        c = lax.axis_index("c")
        s = lax.axis_index("s")
        gid = c * NS + s
        z16 = jnp.zeros((16,), jnp.float32)

        def zr(i, _):
            def zc(j, _):
                rows[0, i, pl.ds(j * 16, 16)] = z16
                return 0
            return lax.fori_loop(0, D // 16, zc, 0)
        lax.fori_loop(0, CH, zr, 0)

        for kk in range(NPT // CH):
            pltpu.sync_copy(rows.at[0], acc.at[pl.ds(s * NPT + kk * CH, CH)])
        plsc.subcore_barrier()

        pltpu.sync_copy(srcp2.at[pl.ds(gid * NCH, NCH)], sidx)
        pltpu.sync_copy(dstp2.at[pl.ds(gid * NCH, NCH)], didx)

        def g_start(k, b):
            pltpu.async_copy(table.at[sidx.at[k]], rows.at[b], gsem.at[b])

        def g_wait(k, b):
            pltpu.make_async_copy(
                table.at[sidx.at[k]], rows.at[b], gsem.at[b]).wait()

        def s_start(k, b):
            pltpu.async_copy(
                rows.at[b], acc.at[didx.at[k]], ssem.at[b], add=True)

        def s_wait(k, b):
            pltpu.make_async_copy(
                rows.at[b], acc.at[didx.at[k]], ssem.at[b]).wait()

        for b in range(NBUF):
            g_start(b, b)

        def step(t, _):
            for b in range(NBUF):
                k = t * NBUF + b
                g_wait(k, b)
                s_start(k, b)
            for b in range(NBUF):
                k = t * NBUF + b
                s_wait(k, b)
                g_start(k + NBUF, b)
            return 0
        lax.fori_loop(0, NCH // NBUF - 1, step, 0)

        t_last = NCH - NBUF
        for b in range(NBUF):
            g_wait(t_last + b, b)
            s_start(t_last + b, b)
        for b in range(NBUF):
            s_wait(t_last + b, b)

        plsc.subcore_barrier()
        for kk in range(NPT // CH):
            r0 = s * NPT + kk * CH
            pltpu.sync_copy(acc.at[pl.ds(r0, CH)], rows.at[0])
            pltpu.sync_copy(rows.at[0], out.at[pl.ds(c * NPAD + r0, CH)])

    return pl.kernel(
        body,
        out_type=jax.ShapeDtypeStruct((2 * NPAD, D), jnp.float32),
        mesh=_mesh(),
        compiler_params=_SC_PARAMS,
        scratch_types=[
            pltpu.VMEM_SHARED((NPAD, D), jnp.float32),
            pltpu.VMEM((NCH, CH), jnp.int32),
            pltpu.VMEM((NCH, CH), jnp.int32),
            pltpu.VMEM((NBUF, CH, D), jnp.float32),
            pltpu.SemaphoreType.DMA((NBUF,)),
            pltpu.SemaphoreType.DMA((NBUF,)),
        ],
    )


EPS2 = 1e-24


def _rownorm2(h):
    d = h.shape[-1]
    j = jnp.ones((d, d), jnp.bfloat16)
    hb = h.astype(jnp.bfloat16)
    return jnp.dot(hb * hb, j, preferred_element_type=jnp.float32)


def _h0_body(x_ref, dgo_ref, o_ref):
    x = x_ref[...]
    dego = jnp.sum(dgo_ref[...], axis=0)
    norm_out = jnp.where(dego > 0, lax.rsqrt(dego), 0.0)
    s = _rownorm2(x)
    o_ref[...] = x * lax.rsqrt(jnp.maximum(s, EPS2)) * norm_out[:, None]


def _h0_call(xp, degp):
    return pl.pallas_call(
        _h0_body,
        grid=(NBLK,),
        in_specs=[
            pl.BlockSpec((RB, DP), lambda i: (i, 0)),
            pl.BlockSpec((2, RB), lambda i: (0, i)),
        ],
        out_specs=pl.BlockSpec((RB, DP), lambda i: (i, 0)),
        out_shape=jax.ShapeDtypeStruct((NPAD, DP), jnp.float32),
    )(xp, degp)


def _l1_body(a0_ref, a1_ref, dgo_ref, dgi_ref, w1_ref, b1_ref, w2_ref, o_ref):
    dego = jnp.sum(dgo_ref[...], axis=0)
    degi = jnp.sum(dgi_ref[...], axis=0)
    no = jnp.where(dego > 0, lax.rsqrt(dego), 0.0)
    ni = jnp.where(degi > 0, lax.rsqrt(degi), 0.0)
    agg = (a0_ref[...] + a1_ref[...]) * ni[:, None]
    h = jnp.dot(agg.astype(jnp.bfloat16), w1_ref[...].astype(jnp.bfloat16),
                preferred_element_type=jnp.float32)
    h = h + b1_ref[...]
    h = h * lax.rsqrt(jnp.maximum(_rownorm2(h), EPS2))
    h = jnp.maximum(h, 0.0)
    h = h * lax.rsqrt(jnp.maximum(_rownorm2(h), EPS2))
    h = h * no[:, None]
    o_ref[...] = jnp.dot(h.astype(jnp.bfloat16),
                         w2_ref[...].astype(jnp.bfloat16),
                         preferred_element_type=jnp.float32)


def _l1_call(agg1, degp, w1p, b1r, w2):
    return pl.pallas_call(
        _l1_body,
        grid=(NBLK,),
        in_specs=[
            pl.BlockSpec((RB, DP), lambda i: (i, 0)),
            pl.BlockSpec((RB, DP), lambda i: (i + NBLK, 0)),
            pl.BlockSpec((2, RB), lambda i: (0, i)),
            pl.BlockSpec((2, RB), lambda i: (0, i + NBLK)),
            pl.BlockSpec((DP, HID), lambda i: (0, 0)),
            pl.BlockSpec((1, HID), lambda i: (0, 0)),
            pl.BlockSpec((HID, OUT2), lambda i: (0, 0)),
        ],
        out_specs=pl.BlockSpec((RB, OUT2), lambda i: (i, 0)),
        out_shape=jax.ShapeDtypeStruct((NPAD, OUT2), jnp.float32),
    )(agg1, agg1, degp, degp, w1p, b1r, w2)


def _l2_body(q0_ref, q1_ref, dgi_ref, b2_ref, wc_ref, bc_ref, o_ref, acc_ref):
    i = pl.program_id(0)
    degi = jnp.sum(dgi_ref[...], axis=0)
    ni = jnp.where(degi > 0, lax.rsqrt(degi), 0.0)
    h = (q0_ref[...] + q1_ref[...]) * ni[:, None] + b2_ref[...]
    h = h * lax.rsqrt(jnp.maximum(_rownorm2(h), EPS2))
    h = jnp.maximum(h, 0.0)
    h = h * lax.rsqrt(jnp.maximum(_rownorm2(h), EPS2))
    rid = lax.broadcasted_iota(jnp.int32, (RB, 1), 0) + i * RB
    h = jnp.where(rid < N, h, 0.0)
    part = jnp.sum(h, axis=0, keepdims=True)

    @pl.when(i == 0)
    def _():
        acc_ref[...] = part

    @pl.when(i > 0)
    def _():
        acc_ref[...] = acc_ref[...] + part

    @pl.when(i == NBLK - 1)
    def _():
        hg = acc_ref[...] / N
        hgn = jnp.sqrt(jnp.sum(hg * hg))
        hg = hg / jnp.maximum(hgn, EPS)
        o_ref[...] = (
            jnp.dot(hg, wc_ref[...], preferred_element_type=jnp.float32)
            + bc_ref[...])


def _l2_call(agg2, degp, b2r, wc, bcr):
    return pl.pallas_call(
        _l2_body,
        grid=(NBLK,),
        in_specs=[
            pl.BlockSpec((RB, OUT2), lambda i: (i, 0)),
            pl.BlockSpec((RB, OUT2), lambda i: (i + NBLK, 0)),
            pl.BlockSpec((2, RB), lambda i: (0, i + NBLK)),
            pl.BlockSpec((1, OUT2), lambda i: (0, 0)),
            pl.BlockSpec((OUT2, NCLS), lambda i: (0, 0)),
            pl.BlockSpec((1, NCLS), lambda i: (0, 0)),
        ],
        out_specs=pl.BlockSpec((1, NCLS), lambda i: (0, 0)),
        out_shape=jax.ShapeDtypeStruct((1, NCLS), jnp.float32),
        scratch_shapes=[pltpu.VMEM((1, OUT2), jnp.float32)],
        compiler_params=pltpu.CompilerParams(
            dimension_semantics=("arbitrary",)),
    )(agg2, agg2, degp, b2r, wc, bcr)


@jax.jit
def kernel(node_embed_weight, edge_index, W1, b1, W2, b2, Wc, bc):
    f32 = jnp.float32
    src = edge_index[0].astype(jnp.int32)
    dst = edge_index[1].astype(jnp.int32)
    npe = EPAD - E
    pad_idx = N + (jnp.arange(npe, dtype=jnp.int32) % (NPAD - N))
    srcp = jnp.concatenate([src, pad_idx]).reshape(EPAD // CH, CH)
    dstp = jnp.concatenate([dst, pad_idx]).reshape(EPAD // CH, CH)

    xp = jnp.zeros((NPAD, DP), f32).at[:N, :IN_DIM].set(
        node_embed_weight.astype(f32))
    w1p = jnp.zeros((DP, HID), f32).at[:IN_DIM].set(W1.astype(f32))

    degp = _deg_kernel()(srcp, dstp).reshape(2, 2 * NPAD)
    h0s = _h0_call(xp, degp)
    agg1 = _make_agg(DP)(h0s, srcp, dstp)
    p = _l1_call(agg1, degp, w1p, b1.reshape(1, HID), W2)
    agg2 = _make_agg(OUT2)(p, srcp, dstp)
    return _l2_call(agg2, degp, b2.reshape(1, OUT2), Wc, bc.reshape(1, NCLS))


# speedup vs baseline: 22.1396x; 1.0975x over previous
import functools

import jax
import jax.numpy as jnp
from jax import lax
from jax.experimental import pallas as pl
from jax.experimental.pallas import tpu as pltpu
from jax.experimental.pallas import tpu_sc as plsc

N = 10000
E = 160000
IN_DIM = 60
DP = 64
HID = 512
OUT2 = 32
NCLS = 16

NC, NS = 2, 16
NW = NC * NS
NPAD = 10240
NPT = NPAD // NS
CH = 128
EW = 5120
EPAD = NW * EW
NCH = EW // CH
RB = 1024
NBLK = NPAD // RB
EPS = 1e-12

@functools.cache
def _mesh():
    return plsc.VectorSubcoreMesh(
        core_axis_name="c", subcore_axis_name="s",
        num_cores=NC, num_subcores=NS)


def _deg_body(srcp2, dstp2, out, acco, acci, sidx, didx, ones, zbuf, dsem):
    c = lax.axis_index("c")
    s = lax.axis_index("s")
    gid = c * NS + s
    z16 = jnp.zeros((16,), jnp.float32)
    o16 = jnp.ones((16,), jnp.float32)
    zn = NPAD // NS

    def zb(i, _):
        zbuf[pl.ds(i * 16, 16)] = z16
        return 0
    lax.fori_loop(0, zn // 16, zb, 0)

    def ob(i, _):
        ones[pl.ds(i * 16, 16)] = o16
        return 0
    lax.fori_loop(0, CH // 16, ob, 0)

    pltpu.sync_copy(zbuf, acco.at[pl.ds(s * zn, zn)])
    pltpu.sync_copy(zbuf, acci.at[pl.ds(s * zn, zn)])
    plsc.subcore_barrier()

    pltpu.sync_copy(srcp2.at[pl.ds(gid * NCH, NCH)], sidx)
    pltpu.sync_copy(dstp2.at[pl.ds(gid * NCH, NCH)], didx)

    FD = 4

    def step(t, _):
        for j in range(FD):
            k = t * FD + j
            pltpu.async_copy(ones, acco.at[sidx.at[k]], dsem, add=True)
            pltpu.async_copy(ones, acci.at[didx.at[k]], dsem, add=True)
        for j in range(FD):
            k = t * FD + j
            pltpu.make_async_copy(ones, acco.at[sidx.at[k]], dsem).wait()
            pltpu.make_async_copy(ones, acci.at[didx.at[k]], dsem).wait()
        return 0
    lax.fori_loop(0, NCH // FD, step, 0)

    plsc.subcore_barrier()
    pltpu.sync_copy(acco.at[pl.ds(s * zn, zn)], zbuf)
    pltpu.sync_copy(zbuf, out.at[pl.ds(c * 2 * NPAD + s * zn, zn)])
    pltpu.sync_copy(acci.at[pl.ds(s * zn, zn)], zbuf)
    pltpu.sync_copy(zbuf, out.at[pl.ds(c * 2 * NPAD + NPAD + s * zn, zn)])


_SC_PARAMS = pltpu.CompilerParams(use_tc_tiling_on_sc=False)


@functools.cache
def _deg_kernel():
    return pl.kernel(
        _deg_body,
        out_type=jax.ShapeDtypeStruct((2 * 2 * NPAD,), jnp.float32),
        mesh=_mesh(),
        compiler_params=_SC_PARAMS,
        scratch_types=[
            pltpu.VMEM_SHARED((NPAD,), jnp.float32),
            pltpu.VMEM_SHARED((NPAD,), jnp.float32),
            pltpu.VMEM((NCH, CH), jnp.int32),
            pltpu.VMEM((NCH, CH), jnp.int32),
            pltpu.VMEM((CH,), jnp.float32),
            pltpu.VMEM((NPAD // NS,), jnp.float32),
            pltpu.SemaphoreType.DMA,
        ],
    )


NBUF = 8


@functools.cache
def _make_agg(D):
    def body(table, srcp2, dstp2, out, acc, sidx, didx, rows, gsem, ssem):
        c = lax.axis_index("c")
        s = lax.axis_index("s")
        gid = c * NS + s
        z32 = jnp.zeros((32,), jnp.bfloat16)

        def zr(i, _):
            def zc(j, _):
                rows[0, i, pl.ds(j * 32, 32)] = z32
                return 0
            return lax.fori_loop(0, D // 32, zc, 0)
        lax.fori_loop(0, CH, zr, 0)

        for kk in range(NPT // CH):
            pltpu.sync_copy(rows.at[0], acc.at[pl.ds(s * NPT + kk * CH, CH)])
        plsc.subcore_barrier()

        pltpu.sync_copy(srcp2.at[pl.ds(gid * NCH, NCH)], sidx)
        pltpu.sync_copy(dstp2.at[pl.ds(gid * NCH, NCH)], didx)

        def g_start(k, b):
            pltpu.async_copy(table.at[sidx.at[k]], rows.at[b], gsem.at[b])

        def g_wait(k, b):
            pltpu.make_async_copy(
                table.at[sidx.at[k]], rows.at[b], gsem.at[b]).wait()

        def s_start(k, b):
            pltpu.async_copy(
                rows.at[b], acc.at[didx.at[k]], ssem.at[b], add=True)

        def s_wait(k, b):
            pltpu.make_async_copy(
                rows.at[b], acc.at[didx.at[k]], ssem.at[b]).wait()

        for b in range(NBUF):
            g_start(b, b)

        def step(t, _):
            for b in range(NBUF):
                k = t * NBUF + b
                g_wait(k, b)
                s_start(k, b)
            for b in range(NBUF):
                k = t * NBUF + b
                s_wait(k, b)
                g_start(k + NBUF, b)
            return 0
        lax.fori_loop(0, NCH // NBUF - 1, step, 0)

        t_last = NCH - NBUF
        for b in range(NBUF):
            g_wait(t_last + b, b)
            s_start(t_last + b, b)
        for b in range(NBUF):
            s_wait(t_last + b, b)

        plsc.subcore_barrier()
        for kk in range(NPT // CH):
            r0 = s * NPT + kk * CH
            pltpu.sync_copy(acc.at[pl.ds(r0, CH)], rows.at[0])
            pltpu.sync_copy(rows.at[0], out.at[pl.ds(c * NPAD + r0, CH)])

    return pl.kernel(
        body,
        out_type=jax.ShapeDtypeStruct((2 * NPAD, D), jnp.bfloat16),
        mesh=_mesh(),
        compiler_params=_SC_PARAMS,
        scratch_types=[
            pltpu.VMEM_SHARED((NPAD, D), jnp.bfloat16),
            pltpu.VMEM((NCH, CH), jnp.int32),
            pltpu.VMEM((NCH, CH), jnp.int32),
            pltpu.VMEM((NBUF, CH, D), jnp.bfloat16),
            pltpu.SemaphoreType.DMA((NBUF,)),
            pltpu.SemaphoreType.DMA((NBUF,)),
        ],
    )


EPS2 = 1e-24


def _rownorm2(h):
    d = h.shape[-1]
    j = jnp.ones((d, d), jnp.bfloat16)
    hb = h.astype(jnp.bfloat16)
    return jnp.dot(hb * hb, j, preferred_element_type=jnp.float32)


def _h0_body(x_ref, dgo_ref, o_ref):
    x = x_ref[...]
    dego = jnp.sum(dgo_ref[...], axis=0)
    norm_out = jnp.where(dego > 0, lax.rsqrt(dego), 0.0)
    s = _rownorm2(x)
    o_ref[...] = (x * lax.rsqrt(jnp.maximum(s, EPS2))
                  * norm_out[:, None]).astype(jnp.bfloat16)


def _h0_call(xp, degp):
    return pl.pallas_call(
        _h0_body,
        grid=(NBLK,),
        in_specs=[
            pl.BlockSpec((RB, DP), lambda i: (i, 0)),
            pl.BlockSpec((2, RB), lambda i: (0, i)),
        ],
        out_specs=pl.BlockSpec((RB, DP), lambda i: (i, 0)),
        out_shape=jax.ShapeDtypeStruct((NPAD, DP), jnp.bfloat16),
    )(xp, degp)


def _l1_body(a0_ref, a1_ref, dgo_ref, dgi_ref, w1_ref, b1_ref, w2_ref, o_ref):
    dego = jnp.sum(dgo_ref[...], axis=0)
    degi = jnp.sum(dgi_ref[...], axis=0)
    no = jnp.where(dego > 0, lax.rsqrt(dego), 0.0)
    ni = jnp.where(degi > 0, lax.rsqrt(degi), 0.0)
    agg = ((a0_ref[...].astype(jnp.float32) + a1_ref[...].astype(jnp.float32))
           * ni[:, None])
    h = jnp.dot(agg.astype(jnp.bfloat16), w1_ref[...].astype(jnp.bfloat16),
                preferred_element_type=jnp.float32)
    h = h + b1_ref[...]
    h = h * lax.rsqrt(jnp.maximum(_rownorm2(h), EPS2))
    h = jnp.maximum(h, 0.0)
    h = h * lax.rsqrt(jnp.maximum(_rownorm2(h), EPS2))
    h = h * no[:, None]
    o_ref[...] = jnp.dot(h.astype(jnp.bfloat16),
                         w2_ref[...].astype(jnp.bfloat16),
                         preferred_element_type=jnp.float32).astype(jnp.bfloat16)


def _l1_call(agg1, degp, w1p, b1r, w2):
    return pl.pallas_call(
        _l1_body,
        grid=(NBLK,),
        in_specs=[
            pl.BlockSpec((RB, DP), lambda i: (i, 0)),
            pl.BlockSpec((RB, DP), lambda i: (i + NBLK, 0)),
            pl.BlockSpec((2, RB), lambda i: (0, i)),
            pl.BlockSpec((2, RB), lambda i: (0, i + NBLK)),
            pl.BlockSpec((DP, HID), lambda i: (0, 0)),
            pl.BlockSpec((1, HID), lambda i: (0, 0)),
            pl.BlockSpec((HID, OUT2), lambda i: (0, 0)),
        ],
        out_specs=pl.BlockSpec((RB, OUT2), lambda i: (i, 0)),
        out_shape=jax.ShapeDtypeStruct((NPAD, OUT2), jnp.bfloat16),
    )(agg1, agg1, degp, degp, w1p, b1r, w2)


def _l2_body(q0_ref, q1_ref, dgi_ref, b2_ref, wc_ref, bc_ref, o_ref, acc_ref):
    i = pl.program_id(0)
    degi = jnp.sum(dgi_ref[...], axis=0)
    ni = jnp.where(degi > 0, lax.rsqrt(degi), 0.0)
    h = ((q0_ref[...].astype(jnp.float32) + q1_ref[...].astype(jnp.float32))
         * ni[:, None] + b2_ref[...])
    h = h * lax.rsqrt(jnp.maximum(_rownorm2(h), EPS2))
    h = jnp.maximum(h, 0.0)
    h = h * lax.rsqrt(jnp.maximum(_rownorm2(h), EPS2))
    rid = lax.broadcasted_iota(jnp.int32, (RB, 1), 0) + i * RB
    h = jnp.where(rid < N, h, 0.0)
    part = jnp.sum(h, axis=0, keepdims=True)

    @pl.when(i == 0)
    def _():
        acc_ref[...] = part

    @pl.when(i > 0)
    def _():
        acc_ref[...] = acc_ref[...] + part

    @pl.when(i == NBLK - 1)
    def _():
        hg = acc_ref[...] / N
        hgn = jnp.sqrt(jnp.sum(hg * hg))
        hg = hg / jnp.maximum(hgn, EPS)
        o_ref[...] = (
            jnp.dot(hg, wc_ref[...], preferred_element_type=jnp.float32)
            + bc_ref[...])


def _l2_call(agg2, degp, b2r, wc, bcr):
    return pl.pallas_call(
        _l2_body,
        grid=(NBLK,),
        in_specs=[
            pl.BlockSpec((RB, OUT2), lambda i: (i, 0)),
            pl.BlockSpec((RB, OUT2), lambda i: (i + NBLK, 0)),
            pl.BlockSpec((2, RB), lambda i: (0, i + NBLK)),
            pl.BlockSpec((1, OUT2), lambda i: (0, 0)),
            pl.BlockSpec((OUT2, NCLS), lambda i: (0, 0)),
            pl.BlockSpec((1, NCLS), lambda i: (0, 0)),
        ],
        out_specs=pl.BlockSpec((1, NCLS), lambda i: (0, 0)),
        out_shape=jax.ShapeDtypeStruct((1, NCLS), jnp.float32),
        scratch_shapes=[pltpu.VMEM((1, OUT2), jnp.float32)],
        compiler_params=pltpu.CompilerParams(
            dimension_semantics=("arbitrary",)),
    )(agg2, agg2, degp, b2r, wc, bcr)


@jax.jit
def kernel(node_embed_weight, edge_index, W1, b1, W2, b2, Wc, bc):
    f32 = jnp.float32
    src = edge_index[0].astype(jnp.int32)
    dst = edge_index[1].astype(jnp.int32)
    npe = EPAD - E
    pad_idx = N + (jnp.arange(npe, dtype=jnp.int32) % (NPAD - N))
    srcp = jnp.concatenate([src, pad_idx]).reshape(EPAD // CH, CH)
    dstp = jnp.concatenate([dst, pad_idx]).reshape(EPAD // CH, CH)

    xp = jnp.zeros((NPAD, DP), f32).at[:N, :IN_DIM].set(
        node_embed_weight.astype(f32))
    w1p = jnp.zeros((DP, HID), f32).at[:IN_DIM].set(W1.astype(f32))

    degp = _deg_kernel()(srcp, dstp).reshape(2, 2 * NPAD)
    h0s = _h0_call(xp, degp)
    agg1 = _make_agg(DP)(h0s, srcp, dstp)
    p = _l1_call(agg1, degp, w1p, b1.reshape(1, HID), W2)
    agg2 = _make_agg(OUT2)(p, srcp, dstp)
    return _l2_call(agg2, degp, b2.reshape(1, OUT2), Wc, bc.reshape(1, NCLS))


# speedup vs baseline: 23.1422x; 1.0453x over previous
import functools

import jax
import jax.numpy as jnp
from jax import lax
from jax.experimental import pallas as pl
from jax.experimental.pallas import tpu as pltpu
from jax.experimental.pallas import tpu_sc as plsc

N = 10000
E = 160000
IN_DIM = 60
DP = 64
HID = 512
OUT2 = 32
NCLS = 16

NC, NS = 2, 16
NW = NC * NS
NPAD = 10240
NPT = NPAD // NS
CH = 128
EW = 5120
EPAD = NW * EW
NCH = EW // CH
RB = 2048
NBLK = NPAD // RB
EPS = 1e-12

@functools.cache
def _mesh():
    return plsc.VectorSubcoreMesh(
        core_axis_name="c", subcore_axis_name="s",
        num_cores=NC, num_subcores=NS)


def _deg_body(srcp2, dstp2, out, acco, acci, sidx, didx, ones, zbuf, dsem):
    c = lax.axis_index("c")
    s = lax.axis_index("s")
    gid = c * NS + s
    z16 = jnp.zeros((16,), jnp.float32)
    o16 = jnp.ones((16,), jnp.float32)
    zn = NPAD // NS

    def zb(i, _):
        zbuf[pl.ds(i * 16, 16)] = z16
        return 0
    lax.fori_loop(0, zn // 16, zb, 0)

    def ob(i, _):
        ones[pl.ds(i * 16, 16)] = o16
        return 0
    lax.fori_loop(0, CH // 16, ob, 0)

    pltpu.sync_copy(zbuf, acco.at[pl.ds(s * zn, zn)])
    pltpu.sync_copy(zbuf, acci.at[pl.ds(s * zn, zn)])
    plsc.subcore_barrier()

    pltpu.sync_copy(srcp2.at[pl.ds(gid * NCH, NCH)], sidx)
    pltpu.sync_copy(dstp2.at[pl.ds(gid * NCH, NCH)], didx)

    FD = 8

    def step(t, _):
        for j in range(FD):
            k = t * FD + j
            pltpu.async_copy(ones, acco.at[sidx.at[k]], dsem, add=True)
            pltpu.async_copy(ones, acci.at[didx.at[k]], dsem, add=True)
        for j in range(FD):
            k = t * FD + j
            pltpu.make_async_copy(ones, acco.at[sidx.at[k]], dsem).wait()
            pltpu.make_async_copy(ones, acci.at[didx.at[k]], dsem).wait()
        return 0
    lax.fori_loop(0, NCH // FD, step, 0)

    plsc.subcore_barrier()
    pltpu.sync_copy(acco.at[pl.ds(s * zn, zn)], zbuf)
    pltpu.sync_copy(zbuf, out.at[pl.ds(c * 2 * NPAD + s * zn, zn)])
    pltpu.sync_copy(acci.at[pl.ds(s * zn, zn)], zbuf)
    pltpu.sync_copy(zbuf, out.at[pl.ds(c * 2 * NPAD + NPAD + s * zn, zn)])


_SC_PARAMS = pltpu.CompilerParams(use_tc_tiling_on_sc=False)


@functools.cache
def _deg_kernel():
    return pl.kernel(
        _deg_body,
        out_type=jax.ShapeDtypeStruct((2 * 2 * NPAD,), jnp.float32),
        mesh=_mesh(),
        compiler_params=_SC_PARAMS,
        scratch_types=[
            pltpu.VMEM_SHARED((NPAD,), jnp.float32),
            pltpu.VMEM_SHARED((NPAD,), jnp.float32),
            pltpu.VMEM((NCH, CH), jnp.int32),
            pltpu.VMEM((NCH, CH), jnp.int32),
            pltpu.VMEM((CH,), jnp.float32),
            pltpu.VMEM((NPAD // NS,), jnp.float32),
            pltpu.SemaphoreType.DMA,
        ],
    )


NBUF = 10


@functools.cache
def _make_agg(D):
    def body(table, srcp2, dstp2, out, acc, sidx, didx, rows, gsem, ssem):
        c = lax.axis_index("c")
        s = lax.axis_index("s")
        gid = c * NS + s
        z32 = jnp.zeros((32,), jnp.bfloat16)

        def zr(i, _):
            def zc(j, _):
                rows[0, i, pl.ds(j * 32, 32)] = z32
                return 0
            return lax.fori_loop(0, D // 32, zc, 0)
        lax.fori_loop(0, CH, zr, 0)

        for kk in range(NPT // CH):
            pltpu.sync_copy(rows.at[0], acc.at[pl.ds(s * NPT + kk * CH, CH)])
        plsc.subcore_barrier()

        pltpu.sync_copy(srcp2.at[pl.ds(gid * NCH, NCH)], sidx)
        pltpu.sync_copy(dstp2.at[pl.ds(gid * NCH, NCH)], didx)

        def g_start(k, b):
            pltpu.async_copy(table.at[sidx.at[k]], rows.at[b], gsem.at[b])

        def g_wait(k, b):
            pltpu.make_async_copy(
                table.at[sidx.at[k]], rows.at[b], gsem.at[b]).wait()

        def s_start(k, b):
            pltpu.async_copy(
                rows.at[b], acc.at[didx.at[k]], ssem.at[b], add=True)

        def s_wait(k, b):
            pltpu.make_async_copy(
                rows.at[b], acc.at[didx.at[k]], ssem.at[b]).wait()

        for b in range(NBUF):
            g_start(b, b)

        def step(t, _):
            for b in range(NBUF):
                k = t * NBUF + b
                g_wait(k, b)
                s_start(k, b)
            for b in range(NBUF):
                k = t * NBUF + b
                s_wait(k, b)
                g_start(k + NBUF, b)
            return 0
        lax.fori_loop(0, NCH // NBUF - 1, step, 0)

        t_last = NCH - NBUF
        for b in range(NBUF):
            g_wait(t_last + b, b)
            s_start(t_last + b, b)
        for b in range(NBUF):
            s_wait(t_last + b, b)

        plsc.subcore_barrier()
        for kk in range(NPT // CH):
            r0 = s * NPT + kk * CH
            pltpu.sync_copy(acc.at[pl.ds(r0, CH)], rows.at[0])
            pltpu.sync_copy(rows.at[0], out.at[pl.ds(c * NPAD + r0, CH)])

    return pl.kernel(
        body,
        out_type=jax.ShapeDtypeStruct((2 * NPAD, D), jnp.bfloat16),
        mesh=_mesh(),
        compiler_params=_SC_PARAMS,
        scratch_types=[
            pltpu.VMEM_SHARED((NPAD, D), jnp.bfloat16),
            pltpu.VMEM((NCH, CH), jnp.int32),
            pltpu.VMEM((NCH, CH), jnp.int32),
            pltpu.VMEM((NBUF, CH, D), jnp.bfloat16),
            pltpu.SemaphoreType.DMA((NBUF,)),
            pltpu.SemaphoreType.DMA((NBUF,)),
        ],
    )


EPS2 = 1e-24


def _rownorm2(h):
    d = h.shape[-1]
    j = jnp.ones((d, d), jnp.bfloat16)
    hb = h.astype(jnp.bfloat16)
    return jnp.dot(hb * hb, j, preferred_element_type=jnp.float32)


def _h0_body(x_ref, dgo_ref, o_ref):
    x = x_ref[...]
    dego = jnp.sum(dgo_ref[...], axis=0)
    norm_out = jnp.where(dego > 0, lax.rsqrt(dego), 0.0)
    s = _rownorm2(x)
    o_ref[...] = (x * lax.rsqrt(jnp.maximum(s, EPS2))
                  * norm_out[:, None]).astype(jnp.bfloat16)


def _h0_call(xp, degp):
    return pl.pallas_call(
        _h0_body,
        grid=(NBLK,),
        in_specs=[
            pl.BlockSpec((RB, DP), lambda i: (i, 0)),
            pl.BlockSpec((2, RB), lambda i: (0, i)),
        ],
        out_specs=pl.BlockSpec((RB, DP), lambda i: (i, 0)),
        out_shape=jax.ShapeDtypeStruct((NPAD, DP), jnp.bfloat16),
    )(xp, degp)


def _l1_body(a0_ref, a1_ref, dgo_ref, dgi_ref, w1_ref, b1_ref, w2_ref, o_ref):
    dego = jnp.sum(dgo_ref[...], axis=0)
    degi = jnp.sum(dgi_ref[...], axis=0)
    no = jnp.where(dego > 0, lax.rsqrt(dego), 0.0)
    ni = jnp.where(degi > 0, lax.rsqrt(degi), 0.0)
    agg = ((a0_ref[...].astype(jnp.float32) + a1_ref[...].astype(jnp.float32))
           * ni[:, None])
    h = jnp.dot(agg.astype(jnp.bfloat16), w1_ref[...].astype(jnp.bfloat16),
                preferred_element_type=jnp.float32)
    h = h + b1_ref[...]
    h = h * lax.rsqrt(jnp.maximum(_rownorm2(h), EPS2))
    h = jnp.maximum(h, 0.0)
    h = h * lax.rsqrt(jnp.maximum(_rownorm2(h), EPS2))
    h = h * no[:, None]
    o_ref[...] = jnp.dot(h.astype(jnp.bfloat16),
                         w2_ref[...].astype(jnp.bfloat16),
                         preferred_element_type=jnp.float32).astype(jnp.bfloat16)


def _l1_call(agg1, degp, w1p, b1r, w2):
    return pl.pallas_call(
        _l1_body,
        grid=(NBLK,),
        in_specs=[
            pl.BlockSpec((RB, DP), lambda i: (i, 0)),
            pl.BlockSpec((RB, DP), lambda i: (i + NBLK, 0)),
            pl.BlockSpec((2, RB), lambda i: (0, i)),
            pl.BlockSpec((2, RB), lambda i: (0, i + NBLK)),
            pl.BlockSpec((DP, HID), lambda i: (0, 0)),
            pl.BlockSpec((1, HID), lambda i: (0, 0)),
            pl.BlockSpec((HID, OUT2), lambda i: (0, 0)),
        ],
        out_specs=pl.BlockSpec((RB, OUT2), lambda i: (i, 0)),
        out_shape=jax.ShapeDtypeStruct((NPAD, OUT2), jnp.bfloat16),
    )(agg1, agg1, degp, degp, w1p, b1r, w2)


def _l2_body(q0_ref, q1_ref, dgi_ref, b2_ref, wc_ref, bc_ref, o_ref, acc_ref):
    i = pl.program_id(0)
    degi = jnp.sum(dgi_ref[...], axis=0)
    ni = jnp.where(degi > 0, lax.rsqrt(degi), 0.0)
    h = ((q0_ref[...].astype(jnp.float32) + q1_ref[...].astype(jnp.float32))
         * ni[:, None] + b2_ref[...])
    h = h * lax.rsqrt(jnp.maximum(_rownorm2(h), EPS2))
    h = jnp.maximum(h, 0.0)
    h = h * lax.rsqrt(jnp.maximum(_rownorm2(h), EPS2))
    rid = lax.broadcasted_iota(jnp.int32, (RB, 1), 0) + i * RB
    h = jnp.where(rid < N, h, 0.0)
    part = jnp.sum(h, axis=0, keepdims=True)

    @pl.when(i == 0)
    def _():
        acc_ref[...] = part

    @pl.when(i > 0)
    def _():
        acc_ref[...] = acc_ref[...] + part

    @pl.when(i == NBLK - 1)
    def _():
        hg = acc_ref[...] / N
        hgn = jnp.sqrt(jnp.sum(hg * hg))
        hg = hg / jnp.maximum(hgn, EPS)
        o_ref[...] = (
            jnp.dot(hg, wc_ref[...], preferred_element_type=jnp.float32)
            + bc_ref[...])


def _l2_call(agg2, degp, b2r, wc, bcr):
    return pl.pallas_call(
        _l2_body,
        grid=(NBLK,),
        in_specs=[
            pl.BlockSpec((RB, OUT2), lambda i: (i, 0)),
            pl.BlockSpec((RB, OUT2), lambda i: (i + NBLK, 0)),
            pl.BlockSpec((2, RB), lambda i: (0, i + NBLK)),
            pl.BlockSpec((1, OUT2), lambda i: (0, 0)),
            pl.BlockSpec((OUT2, NCLS), lambda i: (0, 0)),
            pl.BlockSpec((1, NCLS), lambda i: (0, 0)),
        ],
        out_specs=pl.BlockSpec((1, NCLS), lambda i: (0, 0)),
        out_shape=jax.ShapeDtypeStruct((1, NCLS), jnp.float32),
        scratch_shapes=[pltpu.VMEM((1, OUT2), jnp.float32)],
        compiler_params=pltpu.CompilerParams(
            dimension_semantics=("arbitrary",)),
    )(agg2, agg2, degp, b2r, wc, bcr)


@jax.jit
def kernel(node_embed_weight, edge_index, W1, b1, W2, b2, Wc, bc):
    f32 = jnp.float32
    src = edge_index[0].astype(jnp.int32)
    dst = edge_index[1].astype(jnp.int32)
    npe = EPAD - E
    pad_idx = N + (jnp.arange(npe, dtype=jnp.int32) % (NPAD - N))
    srcp = jnp.concatenate([src, pad_idx]).reshape(EPAD // CH, CH)
    dstp = jnp.concatenate([dst, pad_idx]).reshape(EPAD // CH, CH)

    xp = jnp.zeros((NPAD, DP), f32).at[:N, :IN_DIM].set(
        node_embed_weight.astype(f32))
    w1p = jnp.zeros((DP, HID), f32).at[:IN_DIM].set(W1.astype(f32))

    degp = _deg_kernel()(srcp, dstp).reshape(2, 2 * NPAD)
    h0s = _h0_call(xp, degp)
    agg1 = _make_agg(DP)(h0s, srcp, dstp)
    p = _l1_call(agg1, degp, w1p, b1.reshape(1, HID), W2)
    agg2 = _make_agg(OUT2)(p, srcp, dstp)
    return _l2_call(agg2, degp, b2.reshape(1, OUT2), Wc, bc.reshape(1, NCLS))
